# Initial kernel scaffold; baseline (speedup 1.0000x reference)
import jax
import jax.numpy as jnp
from jax.experimental import pallas as pl


def kernel(edge_index, edge_type, initial_features, relation_embeddings, W1, W2, lin1_w, lin1_b, bn_gamma, bn_beta, lin2_w, lin2_b):
    raise NotImplementedError("write your pallas kernel here")


# speedup vs baseline: 3.1347x; 3.1347x over previous
import functools

import jax
import jax.numpy as jnp
import numpy as np
from jax import lax
from jax.experimental import pallas as pl
from jax.experimental.pallas import tpu as pltpu
from jax.experimental.pallas import tpu_sc as plsc

N = 10000
E = 320000
R = 1001
D = 128

NUM_TILES = 32
C = 64
CHUNKS = -(-E // (C * NUM_TILES))
EPT = CHUNKS * C
E_PAD = EPT * NUM_TILES
N_PAD = 10240
R_PAD = 1008
ROWS_PER_TILE = N_PAD // 16

_mesh = plsc.VectorSubcoreMesh(core_axis_name="c", subcore_axis_name="s")


def _deg_body(dst_hbm, deg_out, dst_v, ones_v, zero_v, deg_sh):
    c = lax.axis_index("c")
    s = lax.axis_index("s")
    wid = s * 2 + c

    zeros16 = jnp.zeros((16,), jnp.float32)
    ones16 = jnp.ones((16,), jnp.float32)

    def fill(i, _):
        for k in range(D // 16):
            ones_v[i, pl.ds(k * 16, 16)] = ones16
            zero_v[i, pl.ds(k * 16, 16)] = zeros16
        return 0
    lax.fori_loop(0, C, fill, 0)

    row0 = s * ROWS_PER_TILE
    for j in range(ROWS_PER_TILE // C):
        pltpu.sync_copy(zero_v, deg_sh.at[pl.ds(row0 + j * C, C)])

    plsc.subcore_barrier()

    base_edge = wid * EPT

    def chunk_body(ci, _):
        off = base_edge + ci * C
        pltpu.sync_copy(dst_hbm.at[pl.ds(off, C)], dst_v)
        pltpu.sync_copy(ones_v, deg_sh.at[dst_v], add=True)
        return 0

    lax.fori_loop(0, CHUNKS, chunk_body, 0)

    plsc.subcore_barrier()

    pltpu.sync_copy(deg_sh.at[pl.ds(row0, ROWS_PER_TILE)],
                    deg_out.at[c, pl.ds(row0, ROWS_PER_TILE)])


_deg_pass = functools.partial(
    pl.kernel,
    out_type=jax.ShapeDtypeStruct((2, N_PAD, D), jnp.float32),
    mesh=_mesh,
    scratch_types=[
        pltpu.VMEM((C,), jnp.int32),
        pltpu.VMEM((C, D), jnp.float32),
        pltpu.VMEM((C, D), jnp.float32),
        pltpu.VMEM_SHARED((N_PAD, D), jnp.float32),
    ],
)(_deg_body)


def _edge_pass_body(x_hbm, p_hbm, src_hbm, typ_hbm, dst_hbm,
                    agg_out,
                    src_v, typ_v, dst_v, h_v, p_v,
                    agg_sh, p_sh, sem_h, sem_p):
    c = lax.axis_index("c")
    s = lax.axis_index("s")
    wid = s * 2 + c

    zeros16 = jnp.zeros((16,), jnp.float32)

    def zrow(i, _):
        for k in range(D // 16):
            h_v[i, pl.ds(k * 16, 16)] = zeros16
        return 0
    lax.fori_loop(0, C, zrow, 0)

    @pl.when(s == 0)
    def _():
        pltpu.sync_copy(p_hbm, p_sh)

    row0 = s * ROWS_PER_TILE
    for j in range(ROWS_PER_TILE // C):
        pltpu.sync_copy(h_v, agg_sh.at[pl.ds(row0 + j * C, C)])

    plsc.subcore_barrier()

    base_edge = wid * EPT
    lanes = lax.iota(jnp.int32, 16)
    _gdn = lax.GatherDimensionNumbers(
        offset_dims=(), collapsed_slice_dims=(0,), start_index_map=(0,))

    def _shuf(v, idx):
        return lax.gather(v, idx[:, None], _gdn, (1,),
                          mode=lax.GatherScatterMode.PROMISE_IN_BOUNDS)

    def chunk_body(ci, _):
        off = base_edge + ci * C
        pltpu.sync_copy(src_hbm.at[pl.ds(off, C)], src_v)
        pltpu.sync_copy(typ_hbm.at[pl.ds(off, C)], typ_v)
        pltpu.sync_copy(dst_hbm.at[pl.ds(off, C)], dst_v)

        cp_h = pltpu.async_copy(x_hbm.at[src_v], h_v, sem_h)
        cp_p = pltpu.async_copy(p_sh.at[typ_v], p_v, sem_p)
        cp_h.wait()
        cp_p.wait()

        def edge_body(e, _):
            acc = zeros16
            for k in range(D // 16):
                acc = acc + h_v[e, pl.ds(k * 16, 16)] * p_v[e, pl.ds(k * 16, 16)]
            for sh in (8, 4, 2, 1):
                acc = acc + _shuf(acc, lanes ^ sh)
            for k in range(D // 16):
                h_v[e, pl.ds(k * 16, 16)] = (
                    h_v[e, pl.ds(k * 16, 16)] - acc * p_v[e, pl.ds(k * 16, 16)]
                )
            return 0
        lax.fori_loop(0, C, edge_body, 0)

        pltpu.sync_copy(h_v, agg_sh.at[dst_v], add=True)
        return 0

    lax.fori_loop(0, CHUNKS, chunk_body, 0)

    plsc.subcore_barrier()

    pltpu.sync_copy(agg_sh.at[pl.ds(row0, ROWS_PER_TILE)],
                    agg_out.at[c, pl.ds(row0, ROWS_PER_TILE)])


_edge_pass = functools.partial(
    pl.kernel,
    out_type=jax.ShapeDtypeStruct((2, N_PAD, D), jnp.float32),
    mesh=_mesh,
    scratch_types=[
        pltpu.VMEM((C,), jnp.int32),
        pltpu.VMEM((C,), jnp.int32),
        pltpu.VMEM((C,), jnp.int32),
        pltpu.VMEM((C, D), jnp.float32),
        pltpu.VMEM((C, D), jnp.float32),
        pltpu.VMEM_SHARED((N_PAD, D), jnp.float32),
        pltpu.VMEM_SHARED((R_PAD, D), jnp.float32),
        pltpu.SemaphoreType.DMA,
        pltpu.SemaphoreType.DMA,
    ],
)(_edge_pass_body)


def _prep_body(r_ref, o_ref):
    r = r_ref[...]
    norm = jnp.sqrt(jnp.sum(r * r, axis=1, keepdims=True))
    o_ref[...] = r * (np.float32(np.sqrt(2.0)) / (norm + 1e-8))


_prep = pl.pallas_call(
    _prep_body,
    out_shape=jax.ShapeDtypeStruct((R_PAD, D), jnp.float32),
)


def _layer_body(a_ref, d_ref, w_ref, o_ref):
    deg = jnp.maximum(d_ref[0, :, 0:1] + d_ref[1, :, 0:1], 1.0)
    x = (a_ref[0] + a_ref[1]) / deg
    o_ref[...] = jnp.maximum(
        jnp.dot(x, w_ref[...], preferred_element_type=jnp.float32), 0.0)


_layer = pl.pallas_call(
    _layer_body,
    out_shape=jax.ShapeDtypeStruct((N_PAD, D), jnp.float32),
)


def _final_body(a_ref, d_ref, w2_ref, l1w_ref, l1b_ref, g_ref, b_ref,
                l2w_ref, l2b_ref, o_ref):
    deg = jnp.maximum(d_ref[0, :, 0:1] + d_ref[1, :, 0:1], 1.0)
    x = (a_ref[0] + a_ref[1]) / deg
    x = jnp.maximum(
        jnp.dot(x, w2_ref[...], preferred_element_type=jnp.float32), 0.0)
    h = jnp.dot(x, l1w_ref[...], preferred_element_type=jnp.float32) + l1b_ref[...]
    mask = (lax.broadcasted_iota(jnp.int32, (N_PAD, 1), 0) < N).astype(jnp.float32)
    cnt = np.float32(N)
    mean = jnp.sum(h * mask, axis=0, keepdims=True) / cnt
    var = jnp.sum((h - mean) ** 2 * mask, axis=0, keepdims=True) / cnt
    h = (h - mean) / jnp.sqrt(var + 1e-5) * g_ref[...] + b_ref[...]
    h = jnp.maximum(h, 0.0)
    o_ref[...] = jnp.dot(h, l2w_ref[...], preferred_element_type=jnp.float32) + l2b_ref[...]


_final = pl.pallas_call(
    _final_body,
    out_shape=jax.ShapeDtypeStruct((N_PAD, D), jnp.float32),
)


def kernel(edge_index, edge_type, initial_features, relation_embeddings,
           W1, W2, lin1_w, lin1_b, bn_gamma, bn_beta, lin2_w, lin2_b):
    pad = E_PAD - E
    src = jnp.concatenate(
        [edge_index[0].astype(jnp.int32), jnp.zeros((pad,), jnp.int32)])
    dst = jnp.concatenate(
        [edge_index[1].astype(jnp.int32), jnp.full((pad,), N, jnp.int32)])
    typ = jnp.concatenate(
        [edge_type.astype(jnp.int32), jnp.zeros((pad,), jnp.int32)])

    x0 = jnp.pad(initial_features, ((0, N_PAD - N), (0, 0)))
    relp = jnp.pad(relation_embeddings, ((0, R_PAD - R), (0, 0)))

    p = _prep(relp)
    deg = _deg_pass(dst)

    agg1 = _edge_pass(x0, p, src, typ, dst)
    x1 = _layer(agg1, deg, W1)
    agg2 = _edge_pass(x1, p, src, typ, dst)
    out = _final(agg2, deg, W2, lin1_w, lin1_b.reshape(1, D),
                 bn_gamma.reshape(1, D), bn_beta.reshape(1, D),
                 lin2_w, lin2_b.reshape(1, D))
    return out[:N]
